# baseline (speedup 1.0000x reference)
import jax
import jax.numpy as jnp
from jax.experimental import pallas as pl

H = 128
N = 10


def _lin(h, p):
    return jnp.dot(h, p["W"], preferred_element_type=jnp.float32) + p["b"]


def _lin_k1(x, p):
    return x * p["W"] + p["b"]


def _bn(h, p):
    m = jnp.mean(h, axis=0, keepdims=True)
    v = jnp.mean((h - m) ** 2, axis=0, keepdims=True)
    return (h - m) / jnp.sqrt(v + 1e-5) * p["g"] + p["b"]


def _genconv(p, x, ea, mask, has_lin):
    if has_lin:
        if x.shape[1] == 1:
            h_src = _lin_k1(x, p["lin_src"])
            x_dst = _lin_k1(x, p["lin_dst"])
        else:
            h_src = _lin(x, p["lin_src"])
            x_dst = _lin(x, p["lin_dst"])
    else:
        h_src = x
        x_dst = x
    we = p["lin_edge"]["W"]
    be = p["lin_edge"]["b"]
    out_rows = []
    for j in range(N):
        ea_j = ea[:, j : j + 1]
        m_j = mask[:, j : j + 1]
        blk = jax.nn.relu(h_src + (ea_j * we + be)) + 1e-7
        blkm = jnp.where(m_j, blk, -jnp.inf)
        mx = blkm[0:1, :]
        for i in range(1, N):
            mx = jnp.maximum(mx, blkm[i : i + 1, :])
        a = jnp.where(m_j, jnp.exp(blk - mx), 0.0)
        am = a * blk
        den = a[0:1, :]
        num = am[0:1, :]
        for i in range(1, N):
            den = den + a[i : i + 1, :]
            num = num + am[i : i + 1, :]
        out_rows.append(num / den)
    out = jnp.concatenate(out_rows, axis=0) + x_dst
    h = _lin(out, p["mlp1"])
    h = jax.nn.relu(_bn(h, p["mlp_bn"]))
    return _lin(h, p["mlp2"])


def _branch(p, feat, ea, mask):
    h = _genconv(p["conv1"], feat, ea, mask, True)
    h = jax.nn.relu(_bn(h, p["bn1"]))
    h = _genconv(p["conv2"], h, ea, mask, False)
    h = jax.nn.relu(_bn(h, p["bn2"]))
    for lp in p["lins"][:-1]:
        h = jax.nn.relu(_lin(h, lp))
    h = _lin(h, p["lins"][-1])
    s = _lin(h, p["att"])
    s = jnp.exp(s - jnp.max(s, axis=0, keepdims=True))
    s = s / jnp.sum(s, axis=0, keepdims=True)
    return jnp.sum(s * h, axis=0, keepdims=True)


def _mlp(ps, h):
    for lp in ps[:-1]:
        h = jax.nn.relu(_lin(h, lp))
    return _lin(h, ps[-1])


def _forward(x, routing, params):
    topo = x[0]
    traf = x[1]
    t_mask = topo != 0.0
    f_mask = traf != 0.0
    t_ea = topo / jnp.sum(topo)
    f_ea = traf / jnp.sum(traf)
    topo_feat = jnp.sum(topo, axis=1, keepdims=True) / jnp.sum(topo)
    traf_feat = jnp.eye(N, dtype=jnp.float32)
    out_t = _branch(params["topology"], topo_feat, t_ea, t_mask)
    out_f = _branch(params["traffic"], traf_feat, f_ea, f_mask)
    out_r = _mlp(params["routing"], routing)
    cat = jnp.concatenate([out_t, out_f, out_r], axis=1)
    return _mlp(params["joint"], cat)


def _fused_body(treedef, x_ref, routing_ref, *refs):
    out_ref = refs[-1]
    leaves = [
        r[...].reshape(1, -1) if len(r.shape) == 1 else r[...] for r in refs[:-1]
    ]
    params = jax.tree_util.tree_unflatten(treedef, leaves)
    out_ref[...] = _forward(x_ref[...], routing_ref[...], params)


def kernel(x, params):
    leaves, treedef = jax.tree_util.tree_flatten(params)
    routing = x[2].reshape(1, N * N)
    body = lambda *refs: _fused_body(treedef, *refs)
    out = pl.pallas_call(
        body,
        out_shape=jax.ShapeDtypeStruct((1, 64), jnp.float32),
    )(x, routing, *leaves)
    return out.reshape(64)


# speedup vs baseline: 1.6349x; 1.6349x over previous
import jax
import jax.numpy as jnp
from jax.experimental import pallas as pl

H = 128
N = 10


def _lin(h, p):
    return jnp.dot(h, p["W"], preferred_element_type=jnp.float32) + p["b"]


def _lin_k1(x, p):
    return x * p["W"] + p["b"]


def _bn(h, p):
    m = jnp.mean(h, axis=0, keepdims=True)
    v = jnp.mean((h - m) ** 2, axis=0, keepdims=True)
    return (h - m) / jnp.sqrt(v + 1e-5) * p["g"] + p["b"]


def _genconv(p, x, ea, mask, has_lin):
    if has_lin:
        if x.shape[1] == 1:
            h_src = _lin_k1(x, p["lin_src"])
            x_dst = _lin_k1(x, p["lin_dst"])
        else:
            h_src = _lin(x, p["lin_src"])
            x_dst = _lin(x, p["lin_dst"])
    else:
        h_src = x
        x_dst = x
    we = p["lin_edge"]["W"]
    be = p["lin_edge"]["b"]
    out_rows = []
    for j in range(N):
        ea_j = ea[:, j : j + 1]
        m_j = mask[:, j : j + 1]
        blk = jax.nn.relu(h_src + (ea_j * we + be)) + 1e-7
        blkm = jnp.where(m_j, blk, -jnp.inf)
        mx = blkm[0:1, :]
        for i in range(1, N):
            mx = jnp.maximum(mx, blkm[i : i + 1, :])
        a = jnp.where(m_j, jnp.exp(blk - mx), 0.0)
        am = a * blk
        den = a[0:1, :]
        num = am[0:1, :]
        for i in range(1, N):
            den = den + a[i : i + 1, :]
            num = num + am[i : i + 1, :]
        out_rows.append(num / den)
    out = jnp.concatenate(out_rows, axis=0) + x_dst
    h = _lin(out, p["mlp1"])
    h = jax.nn.relu(_bn(h, p["mlp_bn"]))
    return _lin(h, p["mlp2"])


def _branch(p, feat, ea, mask):
    h = _genconv(p["conv1"], feat, ea, mask, True)
    h = jax.nn.relu(_bn(h, p["bn1"]))
    h = _genconv(p["conv2"], h, ea, mask, False)
    h = jax.nn.relu(_bn(h, p["bn2"]))
    for lp in p["lins"][:-1]:
        h = jax.nn.relu(_lin(h, lp))
    h = _lin(h, p["lins"][-1])
    w_t = p["att"]["W"]
    hh = h.astype(jnp.bfloat16).astype(jnp.float32)
    hl = (h - hh).astype(jnp.bfloat16).astype(jnp.float32)
    wh = w_t.astype(jnp.bfloat16).astype(jnp.float32)
    wl = (w_t - wh).astype(jnp.bfloat16).astype(jnp.float32)
    prod = hh * wh + (hh * wl + hl * wh)
    s = jnp.sum(prod, axis=1, keepdims=True) + p["att"]["b"]
    s = jnp.exp(s - jnp.max(s, axis=0, keepdims=True))
    s = s / jnp.sum(s, axis=0, keepdims=True)
    return jnp.sum(s * h, axis=0, keepdims=True)


def _lin_t(h, p):
    return (
        jax.lax.dot_general(
            h, p["W"], (((1,), (1,)), ((), ())), preferred_element_type=jnp.float32
        )
        + p["b"]
    )


def _routing_mlp(ps, r2d):
    w0t, b0 = ps[0]["W"], ps[0]["b"]
    h = b0
    for a in range(N):
        h = h + jax.lax.dot_general(
            r2d[a : a + 1, :],
            w0t[:, a * N : (a + 1) * N],
            (((1,), (1,)), ((), ())),
            preferred_element_type=jnp.float32,
        )
    h = jax.nn.relu(h)
    h = jax.nn.relu(_lin(h, ps[1]))
    h = jax.nn.relu(_lin_t(h, ps[2]))
    return _lin(h, ps[3])


def _joint_mlp(ps, h):
    h = jax.nn.relu(_lin(h, ps[0]))
    h = jax.nn.relu(_lin(h, ps[1]))
    return _lin_t(h, ps[2])


def _forward(x, params):
    topo = x[:, 0, :]
    traf = x[:, 1, :]
    r2d = x[:, 2, :]
    t_mask = topo != 0.0
    f_mask = traf != 0.0
    t_ea = topo / jnp.sum(topo)
    f_ea = traf / jnp.sum(traf)
    topo_feat = jnp.sum(topo, axis=1, keepdims=True) / jnp.sum(topo)
    traf_feat = jnp.eye(N, dtype=jnp.float32)
    out_t = _branch(params["topology"], topo_feat, t_ea, t_mask)
    out_f = _branch(params["traffic"], traf_feat, f_ea, f_mask)
    out_r = _routing_mlp(params["routing"], r2d)
    cat = jnp.concatenate([out_t, out_f, out_r], axis=1)
    return _joint_mlp(params["joint"], cat)


def _fused_body(treedef, x_ref, *refs):
    out_ref = refs[-1]
    leaves = [
        r[...].reshape(1, -1) if len(r.shape) == 1 else r[...] for r in refs[:-1]
    ]
    params = jax.tree_util.tree_unflatten(treedef, leaves)
    out_ref[...] = _forward(x_ref[...], params)


def kernel(x, params):
    p2 = jax.tree_util.tree_map(lambda l: l, params)
    p2["topology"]["att"]["W"] = params["topology"]["att"]["W"].T
    p2["traffic"]["att"]["W"] = params["traffic"]["att"]["W"].T
    p2["routing"][0]["W"] = params["routing"][0]["W"].T
    p2["routing"][2]["W"] = params["routing"][2]["W"].T
    p2["joint"][2]["W"] = params["joint"][2]["W"].T
    xt = jnp.transpose(x, (1, 0, 2))
    leaves, treedef = jax.tree_util.tree_flatten(p2)
    body = lambda *refs: _fused_body(treedef, *refs)
    out = pl.pallas_call(
        body,
        out_shape=jax.ShapeDtypeStruct((1, 64), jnp.float32),
    )(xt, *leaves)
    return out.reshape(64)
